# baseline (speedup 1.0000x reference)
import functools

import jax
import jax.numpy as jnp
from jax import lax
from jax.experimental import pallas as pl
from jax.experimental.pallas import tpu as pltpu
from jax.experimental.pallas import tpu_sc as plsc

N = 100000
V = 61
D = 512
CH = 80
NCH = N // CH
NC = 2
NS = 16
NW = NC * NS
MAXCH = (NCH + NW - 1) // NW

_mesh = plsc.VectorSubcoreMesh(core_axis_name="c", subcore_axis_name="s")


@functools.partial(
    pl.kernel,
    out_type=jax.ShapeDtypeStruct((N * D,), jnp.float32),
    mesh=_mesh,
    scratch_types=[
        pltpu.VMEM((MAXCH, CH), jnp.int32),
        pltpu.VMEM((V * D,), jnp.float32),
        pltpu.VMEM((2, CH * D), jnp.float32),
        pltpu.SemaphoreType.DMA((2,)),
    ],
)
def _emb_lookup(idx_hbm, table_hbm, out_hbm, idx_v, table_v, rows_v, ssem):
    wid = lax.axis_index("s") * NC + lax.axis_index("c")
    nchunks = (NCH - wid + NW - 1) // NW

    pltpu.sync_copy(table_hbm, table_v)
    pltpu.sync_copy(idx_hbm.at[:, wid], idx_v)

    def base_of(i):
        return (wid + i * NW) * CH

    def fill_rows(ci, b):
        @plsc.parallel_loop(0, CH // 16)
        def _group(g16):
            r0 = g16 * 16
            idx16 = idx_v[ci, pl.ds(r0, 16)]
            obase = r0 * D
            for j in range(16):
                ibase = idx16[j] * D
                orow = obase + j * D
                vals = [table_v[pl.ds(ibase + c * 16, 16)] for c in range(D // 16)]
                for c in range(D // 16):
                    rows_v[b, pl.ds(orow + c * 16, 16)] = vals[c]

    def start_store(i, b):
        pltpu.make_async_copy(
            rows_v.at[b], out_hbm.at[pl.ds(base_of(i) * D, CH * D)], ssem.at[b]
        ).start()

    def wait_store(b):
        pltpu.make_async_copy(
            rows_v.at[b], out_hbm.at[pl.ds(0, CH * D)], ssem.at[b]
        ).wait()

    def body(g, carry):
        for b in (0, 1):
            i = 2 * g + b

            @pl.when(g > 0)
            def _():
                wait_store(b)

            fill_rows(i, b)
            start_store(i, b)
        return carry

    lax.fori_loop(0, nchunks // 2, body, 0)

    @pl.when(nchunks % 2 == 1)
    def _():
        wait_store(0)
        fill_rows(nchunks - 1, 0)
        start_store(nchunks - 1, 0)

    wait_store(0)
    wait_store(1)


def kernel(x, W0):
    idx = x.reshape(N).astype(jnp.int32)
    idx_pad = jnp.zeros((MAXCH * NW * CH,), jnp.int32).at[:N].set(idx)
    out = _emb_lookup(idx_pad.reshape(MAXCH, NW, CH), W0.reshape(V * D))
    return out.reshape(N, D)


# speedup vs baseline: 2.6415x; 2.6415x over previous
import functools

import jax
import jax.numpy as jnp
from jax import lax
from jax.experimental import pallas as pl
from jax.experimental.pallas import tpu as pltpu
from jax.experimental.pallas import tpu_sc as plsc

N = 100000
V = 61
D = 512
CH = 80
NCH = N // CH
NC = 2
NS = 16
NW = NC * NS
MAXCH = (NCH + NW - 1) // NW

_mesh = plsc.VectorSubcoreMesh(core_axis_name="c", subcore_axis_name="s")


@functools.partial(
    pl.kernel,
    out_type=jax.ShapeDtypeStruct((N, D), jnp.float32),
    mesh=_mesh,
    scratch_types=[
        pltpu.VMEM((MAXCH, CH), jnp.int32),
        pltpu.VMEM((V, D), jnp.float32),
        pltpu.VMEM((2, CH, D), jnp.float32),
        pltpu.SemaphoreType.DMA((2,)),
    ],
)
def _emb_lookup(idx_hbm, table_hbm, out_hbm, idx_v, table_v, rows_v, ssem):
    wid = lax.axis_index("s") * NC + lax.axis_index("c")
    nchunks = (NCH - wid + NW - 1) // NW

    pltpu.sync_copy(table_hbm, table_v)
    pltpu.sync_copy(idx_hbm.at[:, wid], idx_v)

    def base_of(i):
        return (wid + i * NW) * CH

    def fill_rows(ci, b):
        @plsc.parallel_loop(0, CH // 16)
        def _group(g16):
            r0 = g16 * 16
            idx16 = idx_v[ci, pl.ds(r0, 16)]
            for j in range(16):
                row = idx16[j]
                for cb in range(0, D // 16, 8):
                    vals = [table_v[row, pl.ds(c * 16, 16)] for c in range(cb, cb + 8)]
                    for k, c in enumerate(range(cb, cb + 8)):
                        rows_v[b, r0 + j, pl.ds(c * 16, 16)] = vals[k]

    def start_store(i, b):
        pltpu.make_async_copy(
            rows_v.at[b], out_hbm.at[pl.ds(base_of(i), CH)], ssem.at[b]
        ).start()

    def wait_store(b):
        pltpu.make_async_copy(
            rows_v.at[b], out_hbm.at[pl.ds(0, CH)], ssem.at[b]
        ).wait()

    def body(g, carry):
        for b in (0, 1):
            i = 2 * g + b

            @pl.when(g > 0)
            def _():
                wait_store(b)

            fill_rows(i, b)
            start_store(i, b)
        return carry

    lax.fori_loop(0, nchunks // 2, body, 0)

    @pl.when(nchunks % 2 == 1)
    def _():
        wait_store(0)
        fill_rows(nchunks - 1, 0)
        start_store(nchunks - 1, 0)

    wait_store(0)
    wait_store(1)


def kernel(x, W0):
    idx = x.reshape(N).astype(jnp.int32)
    idx_pad = jnp.zeros((MAXCH * NW * CH,), jnp.int32).at[:N].set(idx)
    return _emb_lookup(idx_pad.reshape(MAXCH, NW, CH), W0)
